# baseline (speedup 1.0000x reference)
import functools

import jax
import jax.numpy as jnp
from jax import lax
from jax.experimental import pallas as pl
from jax.experimental.pallas import tpu as pltpu
from jax.experimental.pallas import tpu_sc as plsc

NC = 2
NS = 16
L = 16
NW = NC * NS
C = 128


def _sqrt16(x):
    i = lax.bitcast_convert_type(x, jnp.int32)
    i = jnp.int32(0x5F3759DF) - lax.shift_right_logical(i, 1)
    y = lax.bitcast_convert_type(i, jnp.float32)
    half = x * 0.5
    for _ in range(3):
        y = y * (1.5 - half * y * y)
    return x * y


def kernel(h_batch, t_batch, l_batch, h_apos_batch, t_apos_batch,
           l_apos_batch, E, R, W):
    B = h_batch.shape[0]
    D = E.shape[1]
    f32 = jnp.float32
    b_per_w = B // NW
    nchunk = b_per_w // C
    assert b_per_w * NW == B and nchunk * C == b_per_w and D % L == 0

    mesh = plsc.VectorSubcoreMesh(core_axis_name="c", subcore_axis_name="s")
    vec = jax.ShapeDtypeStruct((B,), f32)
    mat = jax.ShapeDtypeStruct((B, D), f32)

    @functools.partial(
        pl.kernel,
        out_type=(vec, vec, mat, mat, mat, mat),
        mesh=mesh,
        compiler_params=pltpu.CompilerParams(
            needs_layout_passes=False, use_tc_tiling_on_sc=False),
        scratch_types=[
            pltpu.VMEM((C,), jnp.int32),
            pltpu.VMEM((C,), jnp.int32),
            pltpu.VMEM((C,), jnp.int32),
            pltpu.VMEM((C, D), f32),
            pltpu.VMEM((C, D), f32),
            pltpu.VMEM((C, D), f32),
            pltpu.VMEM((C, D), f32),
            pltpu.VMEM((C, D), f32),
            pltpu.VMEM((C, D), f32),
            pltpu.VMEM((C,), f32),
            pltpu.SemaphoreType.DMA,
        ],
    )
    def run(h_i, t_i, l_i, ha_i, ta_i, la_i, E_h, R_h, W_h,
            dist_o, dista_o, hp_o, tp_o, hpa_o, tpa_o,
            hi_v, ti_v, li_v, hr, tr, lr, wr, hp_v, tp_v, dist_v, sem):
        wid = lax.axis_index("s") * NC + lax.axis_index("c")
        zero = jnp.zeros((L,), f32)
        for (hb, tb, lb, d_o, hpo, tpo) in (
            (h_i, t_i, l_i, dist_o, hp_o, tp_o),
            (ha_i, ta_i, la_i, dista_o, hpa_o, tpa_o),
        ):
            def chunk_body(c, carry, hb=hb, tb=tb, lb=lb, d_o=d_o,
                           hpo=hpo, tpo=tpo):
                base = wid * b_per_w + c * C
                pltpu.sync_copy(hb.at[pl.ds(base, C)], hi_v)
                pltpu.sync_copy(tb.at[pl.ds(base, C)], ti_v)
                pltpu.sync_copy(lb.at[pl.ds(base, C)], li_v)
                cps = [
                    pltpu.async_copy(E_h.at[hi_v], hr, sem),
                    pltpu.async_copy(E_h.at[ti_v], tr, sem),
                    pltpu.async_copy(R_h.at[li_v], lr, sem),
                    pltpu.async_copy(W_h.at[li_v], wr, sem),
                ]
                for cp in cps:
                    cp.wait()

                def group_body(g, gcarry):
                    rows = g * L + lax.iota(jnp.int32, L)

                    def dot_d(d, dcarry):
                        n2, sh, st = dcarry
                        col = jnp.full((L,), d, jnp.int32)
                        wv = plsc.load_gather(wr, [rows, col])
                        hv = plsc.load_gather(hr, [rows, col])
                        tv = plsc.load_gather(tr, [rows, col])
                        return (n2 + wv * wv, sh + hv * wv, st + tv * wv)

                    n2, sh, st = lax.fori_loop(0, D, dot_d,
                                               (zero, zero, zero))
                    ah = sh / n2
                    atc = st / n2

                    def out_d(d, acc):
                        col = jnp.full((L,), d, jnp.int32)
                        wv = plsc.load_gather(wr, [rows, col])
                        hv = plsc.load_gather(hr, [rows, col])
                        tv = plsc.load_gather(tr, [rows, col])
                        lv = plsc.load_gather(lr, [rows, col])
                        hp = hv - ah * wv
                        tp = tv - atc * wv
                        plsc.store_scatter(hp_v, [rows, col], hp)
                        plsc.store_scatter(tp_v, [rows, col], tp)
                        dv = hp + lv - tp
                        return acc + dv * dv

                    acc = lax.fori_loop(0, D, out_d, zero)
                    dist_v[pl.ds(g * L, L)] = _sqrt16(acc)
                    return gcarry

                lax.fori_loop(0, C // L, group_body, 0)
                pltpu.sync_copy(hp_v, hpo.at[pl.ds(base, C)])
                pltpu.sync_copy(tp_v, tpo.at[pl.ds(base, C)])
                pltpu.sync_copy(dist_v, d_o.at[pl.ds(base, C)])
                return carry

            lax.fori_loop(0, nchunk, chunk_body, 0)

    return run(h_batch.astype(jnp.int32), t_batch.astype(jnp.int32),
               l_batch.astype(jnp.int32), h_apos_batch.astype(jnp.int32),
               t_apos_batch.astype(jnp.int32), l_apos_batch.astype(jnp.int32),
               E, R, W)


# speedup vs baseline: 1.1323x; 1.1323x over previous
import functools

import jax
import jax.numpy as jnp
from jax import lax
from jax.experimental import pallas as pl
from jax.experimental.pallas import tpu as pltpu
from jax.experimental.pallas import tpu_sc as plsc

NC = 2
NS = 16
L = 16
NW = NC * NS
C = 128


def _sqrt16(x):
    i = lax.bitcast_convert_type(x, jnp.int32)
    i = jnp.int32(0x5F3759DF) - lax.shift_right_logical(i, 1)
    y = lax.bitcast_convert_type(i, jnp.float32)
    half = x * 0.5
    for _ in range(3):
        y = y * (1.5 - half * y * y)
    return x * y


def kernel(h_batch, t_batch, l_batch, h_apos_batch, t_apos_batch,
           l_apos_batch, E, R, W):
    B = h_batch.shape[0]
    D = E.shape[1]
    f32 = jnp.float32
    b_per_w = B // NW
    nchunk = b_per_w // C
    assert b_per_w * NW == B and nchunk * C == b_per_w and D % L == 0

    mesh = plsc.VectorSubcoreMesh(core_axis_name="c", subcore_axis_name="s")
    vec = jax.ShapeDtypeStruct((B,), f32)
    mat = jax.ShapeDtypeStruct((B, D), f32)

    idx_t = pltpu.VMEM((C,), jnp.int32)
    row_t = pltpu.VMEM((C, D), f32)

    @functools.partial(
        pl.kernel,
        out_type=(vec, vec, mat, mat, mat, mat),
        mesh=mesh,
        compiler_params=pltpu.CompilerParams(
            needs_layout_passes=False, use_tc_tiling_on_sc=False),
        scratch_types=[
            [idx_t] * 2, [idx_t] * 2, [idx_t] * 2,
            [row_t] * 2, [row_t] * 2, [row_t] * 2, [row_t] * 2,
            [row_t] * 2, [row_t] * 2,
            [pltpu.VMEM((C,), f32)] * 2,
            pltpu.SemaphoreType.DMA,
            pltpu.SemaphoreType.DMA,
        ],
    )
    def run(h_i, t_i, l_i, ha_i, ta_i, la_i, E_h, R_h, W_h,
            dist_o, dista_o, hp_o, tp_o, hpa_o, tpa_o,
            hi_v, ti_v, li_v, hr, tr, lr, wr, hp_v, tp_v, dist_v,
            gsem, ssem):
        wid = lax.axis_index("s") * NC + lax.axis_index("c")
        zero = jnp.zeros((L,), f32)

        sides = (
            (h_i, t_i, l_i, dist_o, hp_o, tp_o),
            (ha_i, ta_i, la_i, dista_o, hpa_o, tpa_o),
        )
        tasks = [(s, c) for s in range(2) for c in range(nchunk)]

        def start_gathers(task, slot):
            s, c = task
            hb, tb, lb, _, _, _ = sides[s]
            base = wid * b_per_w + c * C
            pltpu.sync_copy(hb.at[pl.ds(base, C)], hi_v[slot])
            pltpu.sync_copy(tb.at[pl.ds(base, C)], ti_v[slot])
            pltpu.sync_copy(lb.at[pl.ds(base, C)], li_v[slot])
            return [
                pltpu.async_copy(E_h.at[hi_v[slot]], hr[slot], gsem),
                pltpu.async_copy(E_h.at[ti_v[slot]], tr[slot], gsem),
                pltpu.async_copy(R_h.at[li_v[slot]], lr[slot], gsem),
                pltpu.async_copy(W_h.at[li_v[slot]], wr[slot], gsem),
            ]

        def compute(slot):
            hrs, trs, lrs, wrs = hr[slot], tr[slot], lr[slot], wr[slot]
            hps, tps, dv = hp_v[slot], tp_v[slot], dist_v[slot]

            @plsc.parallel_loop(0, C // L)
            def _group(g):
                rows = g * L + lax.iota(jnp.int32, L)

                @plsc.parallel_loop(0, D, unroll=8, carry=(zero, zero, zero))
                def dots(d, dcarry):
                    n2, sh, st = dcarry
                    col = jnp.full((L,), d, jnp.int32)
                    wv = plsc.load_gather(wrs, [rows, col])
                    hv = plsc.load_gather(hrs, [rows, col])
                    tv = plsc.load_gather(trs, [rows, col])
                    return (n2 + wv * wv, sh + hv * wv, st + tv * wv)

                n2, sh, st = dots
                ah = sh / n2
                atc = st / n2

                @plsc.parallel_loop(0, D, unroll=8, carry=zero)
                def accd(d, acc):
                    col = jnp.full((L,), d, jnp.int32)
                    wv = plsc.load_gather(wrs, [rows, col])
                    hv = plsc.load_gather(hrs, [rows, col])
                    tv = plsc.load_gather(trs, [rows, col])
                    lv = plsc.load_gather(lrs, [rows, col])
                    hp = hv - ah * wv
                    tp = tv - atc * wv
                    plsc.store_scatter(hps, [rows, col], hp)
                    plsc.store_scatter(tps, [rows, col], tp)
                    dvv = hp + lv - tp
                    return acc + dvv * dvv

                dv[pl.ds(g * L, L)] = _sqrt16(accd)

        def start_stores(task, slot):
            s, c = task
            _, _, _, d_o, hpo, tpo = sides[s]
            base = wid * b_per_w + c * C
            return [
                pltpu.async_copy(hp_v[slot], hpo.at[pl.ds(base, C)], ssem),
                pltpu.async_copy(tp_v[slot], tpo.at[pl.ds(base, C)], ssem),
                pltpu.async_copy(dist_v[slot], d_o.at[pl.ds(base, C)], ssem),
            ]

        pending_g = start_gathers(tasks[0], 0)
        pending_s = []
        for i, task in enumerate(tasks):
            slot = i % 2
            for cp in pending_g:
                cp.wait()
            if i + 1 < len(tasks):
                pending_g = start_gathers(tasks[i + 1], (i + 1) % 2)
            for cp in pending_s:
                cp.wait()
            compute(slot)
            pending_s = start_stores(task, slot)
        for cp in pending_s:
            cp.wait()

    return run(h_batch.astype(jnp.int32), t_batch.astype(jnp.int32),
               l_batch.astype(jnp.int32), h_apos_batch.astype(jnp.int32),
               t_apos_batch.astype(jnp.int32), l_apos_batch.astype(jnp.int32),
               E, R, W)


# speedup vs baseline: 1.1860x; 1.0474x over previous
import functools

import jax
import jax.numpy as jnp
from jax import lax
from jax.experimental import pallas as pl
from jax.experimental.pallas import tpu as pltpu
from jax.experimental.pallas import tpu_sc as plsc

NC = 2
NS = 16
L = 16
NW = NC * NS
C = 128


def _sqrt16(x):
    i = lax.bitcast_convert_type(x, jnp.int32)
    i = jnp.int32(0x5F3759DF) - lax.shift_right_logical(i, 1)
    y = lax.bitcast_convert_type(i, jnp.float32)
    half = x * 0.5
    for _ in range(3):
        y = y * (1.5 - half * y * y)
    return x * y


def kernel(h_batch, t_batch, l_batch, h_apos_batch, t_apos_batch,
           l_apos_batch, E, R, W):
    B = h_batch.shape[0]
    D = E.shape[1]
    NR = R.shape[0]
    f32 = jnp.float32
    b_per_w = B // NW
    nchunk = b_per_w // C
    assert b_per_w * NW == B and nchunk * C == b_per_w and D % L == 0

    mesh = plsc.VectorSubcoreMesh(core_axis_name="c", subcore_axis_name="s")
    vec = jax.ShapeDtypeStruct((B,), f32)
    matT = jax.ShapeDtypeStruct((D, B), f32)

    idx_t = pltpu.VMEM((C,), jnp.int32)
    row_t = pltpu.VMEM((C, D), f32)
    out_t = pltpu.VMEM((D, C), f32)

    @functools.partial(
        pl.kernel,
        out_type=(vec, vec, matT, matT, matT, matT),
        mesh=mesh,
        compiler_params=pltpu.CompilerParams(
            needs_layout_passes=False, use_tc_tiling_on_sc=False),
        scratch_types=[
            [idx_t] * 2, [idx_t] * 2, [idx_t] * 2,
            [row_t] * 2, [row_t] * 2, [row_t] * 2, [row_t] * 2,
            [out_t] * 2, [out_t] * 2,
            [pltpu.VMEM((C,), f32)] * 2,
            pltpu.SemaphoreType.DMA,
            pltpu.SemaphoreType.DMA,
        ],
    )
    def run(h_i, t_i, l_i, ha_i, ta_i, la_i, E_h, R_h, W_h,
            dist_o, dista_o, hp_o, tp_o, hpa_o, tpa_o,
            hi_v, ti_v, li_v, hr, tr, lr, wr, hp_v, tp_v, dist_v,
            gsem, ssem):
        cid = lax.axis_index("c")
        sid = lax.axis_index("s")
        wid = sid * NC + cid
        zero = jnp.zeros((L,), f32)

        sides = (
            (h_i, t_i, l_i, dist_o, hp_o, tp_o),
            (ha_i, ta_i, la_i, dista_o, hpa_o, tpa_o),
        )
        tasks = [(s, c) for s in range(2) for c in range(nchunk)]

        def start_gathers(task, slot):
            s, c = task
            hb, tb, lb, _, _, _ = sides[s]
            base = wid * b_per_w + c * C
            pltpu.sync_copy(hb.at[pl.ds(base, C)], hi_v[slot])
            pltpu.sync_copy(tb.at[pl.ds(base, C)], ti_v[slot])
            pltpu.sync_copy(lb.at[pl.ds(base, C)], li_v[slot])
            return [
                pltpu.async_copy(E_h.at[hi_v[slot]], hr[slot], gsem),
                pltpu.async_copy(E_h.at[ti_v[slot]], tr[slot], gsem),
                pltpu.async_copy(R_h.at[li_v[slot]], lr[slot], gsem),
                pltpu.async_copy(W_h.at[li_v[slot]], wr[slot], gsem),
            ]

        def compute(slot):
            hrs, trs, lrs, wrs = hr[slot], tr[slot], lr[slot], wr[slot]
            hps, tps, dv = hp_v[slot], tp_v[slot], dist_v[slot]

            @plsc.parallel_loop(0, C // L)
            def _group(g):
                rows = g * L + lax.iota(jnp.int32, L)

                @plsc.parallel_loop(0, D, unroll=8, carry=(zero, zero, zero))
                def dots(d, dcarry):
                    n2, sh, st = dcarry
                    col = jnp.full((L,), d, jnp.int32)
                    wv = plsc.load_gather(wrs, [rows, col])
                    hv = plsc.load_gather(hrs, [rows, col])
                    tv = plsc.load_gather(trs, [rows, col])
                    return (n2 + wv * wv, sh + hv * wv, st + tv * wv)

                n2, sh, st = dots
                ah = sh / n2
                atc = st / n2

                @plsc.parallel_loop(0, D, unroll=8, carry=zero)
                def accd(d, acc):
                    col = jnp.full((L,), d, jnp.int32)
                    wv = plsc.load_gather(wrs, [rows, col])
                    hv = plsc.load_gather(hrs, [rows, col])
                    tv = plsc.load_gather(trs, [rows, col])
                    lv = plsc.load_gather(lrs, [rows, col])
                    hp = hv - ah * wv
                    tp = tv - atc * wv
                    hps[d, pl.ds(g * L, L)] = hp
                    tps[d, pl.ds(g * L, L)] = tp
                    dvv = hp + lv - tp
                    return acc + dvv * dvv

                dv[pl.ds(g * L, L)] = _sqrt16(accd)

        def start_stores(task, slot):
            s, c = task
            _, _, _, d_o, hpo, tpo = sides[s]
            base = wid * b_per_w + c * C
            return [
                pltpu.async_copy(hp_v[slot], hpo.at[:, pl.ds(base, C)], ssem),
                pltpu.async_copy(tp_v[slot], tpo.at[:, pl.ds(base, C)], ssem),
                pltpu.async_copy(dist_v[slot], d_o.at[pl.ds(base, C)], ssem),
            ]

        pending_g = start_gathers(tasks[0], 0)
        pending_s = []
        for i, task in enumerate(tasks):
            slot = i % 2
            for cp in pending_g:
                cp.wait()
            if i + 1 < len(tasks):
                pending_g = start_gathers(tasks[i + 1], (i + 1) % 2)
            for cp in pending_s:
                cp.wait()
            compute(slot)
            pending_s = start_stores(task, slot)
        for cp in pending_s:
            cp.wait()

    dist, dist_a, hpT, tpT, hpaT, tpaT = run(
        h_batch.astype(jnp.int32), t_batch.astype(jnp.int32),
        l_batch.astype(jnp.int32), h_apos_batch.astype(jnp.int32),
        t_apos_batch.astype(jnp.int32), l_apos_batch.astype(jnp.int32),
        E, R, W)
    return (dist, dist_a, jnp.transpose(hpT), jnp.transpose(tpT),
            jnp.transpose(hpaT), jnp.transpose(tpaT))
